# Initial kernel scaffold; baseline (speedup 1.0000x reference)
import jax
import jax.numpy as jnp
from jax.experimental import pallas as pl


def kernel(user, item, theta_w, a_w, b_w, c_w):
    raise NotImplementedError("write your pallas kernel here")


# speedup vs baseline: 3.3032x; 3.3032x over previous
import functools

import jax
import jax.numpy as jnp
from jax import lax
from jax.experimental import pallas as pl
from jax.experimental.pallas import tpu as pltpu
from jax.experimental.pallas import tpu_sc as plsc

BATCH = 16384
USERS = 1000000
ITEMS = 100000

NC = 2
NS = 16
L = 16
NW = NC * NS
BPW = BATCH // NW
CHUNK = 128
NCHUNK = BPW // CHUNK
NV = BPW // L

_mesh = plsc.VectorSubcoreMesh(core_axis_name="c", subcore_axis_name="s")


def _softplus(x):
    t = jnp.exp(-jnp.abs(x))
    g = t * (0.99930147 + t * (-0.48463644 + t * (0.25187601 + t * -0.07389941)))
    g = g + (1.0 + t) * jnp.exp(-g) - 1.0
    return jnp.maximum(x, 0.0) + g


@functools.partial(
    pl.kernel,
    out_type=jax.ShapeDtypeStruct((BATCH,), jnp.float32),
    mesh=_mesh,
    scratch_types=[
        pltpu.VMEM((NCHUNK, CHUNK), jnp.int32),
        pltpu.VMEM((NCHUNK, CHUNK), jnp.int32),
        pltpu.VMEM((BPW,), jnp.float32),
        pltpu.VMEM((BPW,), jnp.float32),
        pltpu.VMEM((BPW,), jnp.float32),
        pltpu.VMEM((BPW,), jnp.float32),
        pltpu.VMEM((BPW,), jnp.float32),
        pltpu.SemaphoreType.DMA,
    ],
)
def _irt_sc(user_hbm, item_hbm, theta_hbm, a_hbm, b_hbm, c_hbm, out_hbm,
            uidx, iidx, tv, av, bv, cv, ov, sem):
    wid = lax.axis_index("s") * NC + lax.axis_index("c")
    base = wid * BPW

    theta_flat = theta_hbm.at[0]
    a_flat = a_hbm.at[0]
    b_flat = b_hbm.at[0]
    c_flat = c_hbm.at[0]

    idx_copies = []
    for j in range(NCHUNK):
        sl = pl.ds(base + j * CHUNK, CHUNK)
        idx_copies.append(pltpu.make_async_copy(user_hbm.at[sl], uidx.at[j], sem))
        idx_copies.append(pltpu.make_async_copy(item_hbm.at[sl], iidx.at[j], sem))
    for cpy in idx_copies:
        cpy.start()
    for cpy in idx_copies:
        cpy.wait()

    copies = []
    for j in range(NCHUNK):
        sl = pl.ds(j * CHUNK, CHUNK)
        copies.append(pltpu.make_async_copy(theta_flat.at[uidx.at[j]], tv.at[sl], sem))
        copies.append(pltpu.make_async_copy(a_flat.at[iidx.at[j]], av.at[sl], sem))
        copies.append(pltpu.make_async_copy(b_flat.at[iidx.at[j]], bv.at[sl], sem))
        copies.append(pltpu.make_async_copy(c_flat.at[iidx.at[j]], cv.at[sl], sem))
    for cpy in copies:
        cpy.start()
    for cpy in copies:
        cpy.wait()

    for i in range(NV):
        s = pl.ds(i * L, L)
        theta = tv[s]
        a = _softplus(av[s])
        b = bv[s]
        c = 1.0 / (1.0 + jnp.exp(-cv[s]))
        ov[s] = c + (1.0 - c) / (1.0 + jnp.exp(-a * (theta - b)))

    pltpu.sync_copy(ov, out_hbm.at[pl.ds(base, BPW)])


def kernel(user, item, theta_w, a_w, b_w, c_w):
    return _irt_sc(
        user,
        item,
        theta_w.reshape(1, USERS),
        a_w.reshape(1, ITEMS),
        b_w.reshape(1, ITEMS),
        c_w.reshape(1, ITEMS),
    )


# speedup vs baseline: 3.3840x; 1.0245x over previous
import functools

import jax
import jax.numpy as jnp
from jax import lax
from jax.experimental import pallas as pl
from jax.experimental.pallas import tpu as pltpu
from jax.experimental.pallas import tpu_sc as plsc

BATCH = 16384
USERS = 1000000
ITEMS = 100000

NC = 2
NS = 16
L = 16
NW = NC * NS
BPW = BATCH // NW
CHUNK = 128
NCHUNK = BPW // CHUNK
VPC = CHUNK // L

_mesh = plsc.VectorSubcoreMesh(core_axis_name="c", subcore_axis_name="s")


def _softplus(x):
    t = jnp.exp(-jnp.abs(x))
    g = t * (0.99930147 + t * (-0.48463644 + t * (0.25187601 + t * -0.07389941)))
    g = g + (1.0 + t) * jnp.exp(-g) - 1.0
    return jnp.maximum(x, 0.0) + g


@functools.partial(
    pl.kernel,
    out_type=jax.ShapeDtypeStruct((BATCH,), jnp.float32),
    mesh=_mesh,
    scratch_types=[
        pltpu.VMEM((BPW,), jnp.int32),
        pltpu.VMEM((BPW,), jnp.int32),
        pltpu.VMEM((BPW,), jnp.float32),
        pltpu.VMEM((BPW,), jnp.float32),
        pltpu.VMEM((BPW,), jnp.float32),
        pltpu.VMEM((BPW,), jnp.float32),
        pltpu.VMEM((BPW,), jnp.float32),
        pltpu.SemaphoreType.DMA,
        pltpu.SemaphoreType.DMA,
        pltpu.SemaphoreType.DMA,
        pltpu.SemaphoreType.DMA,
        pltpu.SemaphoreType.DMA,
    ],
)
def _irt_sc(user_hbm, item_hbm, theta_hbm, a_hbm, b_hbm, c_hbm, out_hbm,
            uidx, iidx, tv, av, bv, cv, ov, sem, g0, g1, g2, g3):
    wid = lax.axis_index("s") * NC + lax.axis_index("c")
    base = wid * BPW
    gsems = (g0, g1, g2, g3)

    theta_flat = theta_hbm.at[0]
    a_flat = a_hbm.at[0]
    b_flat = b_hbm.at[0]
    c_flat = c_hbm.at[0]

    iu = pltpu.make_async_copy(user_hbm.at[pl.ds(base, BPW)], uidx, sem)
    ii = pltpu.make_async_copy(item_hbm.at[pl.ds(base, BPW)], iidx, sem)
    iu.start()
    ii.start()
    iu.wait()
    ii.wait()

    groups = []
    for j in range(NCHUNK):
        sl = pl.ds(j * CHUNK, CHUNK)
        grp = (
            pltpu.make_async_copy(theta_flat.at[uidx.at[sl]], tv.at[sl], gsems[j]),
            pltpu.make_async_copy(a_flat.at[iidx.at[sl]], av.at[sl], gsems[j]),
            pltpu.make_async_copy(b_flat.at[iidx.at[sl]], bv.at[sl], gsems[j]),
            pltpu.make_async_copy(c_flat.at[iidx.at[sl]], cv.at[sl], gsems[j]),
        )
        for cpy in grp:
            cpy.start()
        groups.append(grp)

    for j in range(NCHUNK):
        for cpy in groups[j]:
            cpy.wait()
        for i in range(j * VPC, (j + 1) * VPC):
            s = pl.ds(i * L, L)
            theta = tv[s]
            a = _softplus(av[s])
            b = bv[s]
            c = 1.0 / (1.0 + jnp.exp(-cv[s]))
            ov[s] = c + (1.0 - c) / (1.0 + jnp.exp(-a * (theta - b)))

    pltpu.sync_copy(ov, out_hbm.at[pl.ds(base, BPW)])


def kernel(user, item, theta_w, a_w, b_w, c_w):
    return _irt_sc(
        user,
        item,
        theta_w.reshape(1, USERS),
        a_w.reshape(1, ITEMS),
        b_w.reshape(1, ITEMS),
        c_w.reshape(1, ITEMS),
    )


# speedup vs baseline: 3.6711x; 1.0848x over previous
import functools

import jax
import jax.numpy as jnp
from jax import lax
from jax.experimental import pallas as pl
from jax.experimental.pallas import tpu as pltpu
from jax.experimental.pallas import tpu_sc as plsc

BATCH = 16384
USERS = 1000000
ITEMS = 100000

NC = 2
NS = 16
L = 16
NW = NC * NS
BPW = BATCH // NW
CHUNK = 128
NCHUNK = BPW // CHUNK
VPC = CHUNK // L

_mesh = plsc.VectorSubcoreMesh(core_axis_name="c", subcore_axis_name="s")


def _softplus(x):
    t = jnp.exp(-jnp.abs(x))
    g = t * (0.99930147 + t * (-0.48463644 + t * (0.25187601 + t * -0.07389941)))
    g = g + (1.0 + t) * jnp.exp(-g) - 1.0
    return jnp.maximum(x, 0.0) + g


@functools.partial(
    pl.kernel,
    out_type=jax.ShapeDtypeStruct((BATCH,), jnp.float32),
    mesh=_mesh,
    scratch_types=[
        pltpu.VMEM((BPW,), jnp.int32),
        pltpu.VMEM((BPW,), jnp.int32),
        pltpu.VMEM((BPW,), jnp.float32),
        pltpu.VMEM((BPW,), jnp.float32),
        pltpu.VMEM((BPW,), jnp.float32),
        pltpu.VMEM((BPW,), jnp.float32),
        pltpu.VMEM((BPW,), jnp.float32),
        pltpu.SemaphoreType.DMA,
        pltpu.SemaphoreType.DMA,
        pltpu.SemaphoreType.DMA,
        pltpu.SemaphoreType.DMA,
        pltpu.SemaphoreType.DMA,
    ],
)
def _irt_sc(user_hbm, item_hbm, theta_hbm, a_hbm, b_hbm, c_hbm, out_hbm,
            uidx, iidx, tv, av, bv, cv, ov, sem, g0, g1, g2, g3):
    wid = lax.axis_index("s") * NC + lax.axis_index("c")
    base = wid * BPW
    gsems = (g0, g1, g2, g3)

    theta_flat = theta_hbm.at[0]
    a_flat = a_hbm.at[0]
    b_flat = b_hbm.at[0]
    c_flat = c_hbm.at[0]

    iu = pltpu.make_async_copy(user_hbm.at[pl.ds(base, BPW)], uidx, sem)
    ii = pltpu.make_async_copy(item_hbm.at[pl.ds(base, BPW)], iidx, sem)
    iu.start()
    ii.start()
    iu.wait()
    ii.wait()

    groups = []
    for j in range(NCHUNK):
        sl = pl.ds(j * CHUNK, CHUNK)
        grp = (
            pltpu.make_async_copy(theta_flat.at[uidx.at[sl]], tv.at[sl], gsems[j]),
            pltpu.make_async_copy(a_flat.at[iidx.at[sl]], av.at[sl], gsems[j]),
            pltpu.make_async_copy(b_flat.at[iidx.at[sl]], bv.at[sl], gsems[j]),
            pltpu.make_async_copy(c_flat.at[iidx.at[sl]], cv.at[sl], gsems[j]),
        )
        for cpy in grp:
            cpy.start()
        groups.append(grp)

    def body(i, _):
        s = pl.ds(i * L, L)
        theta = tv[s]
        a = _softplus(av[s])
        b = bv[s]
        c = 1.0 / (1.0 + jnp.exp(-cv[s]))
        ov[s] = c + (1.0 - c) / (1.0 + jnp.exp(-a * (theta - b)))
        return _

    out_copies = []
    for j in range(NCHUNK):
        for cpy in groups[j]:
            cpy.wait()
        lax.fori_loop(j * VPC, (j + 1) * VPC, body, 0)
        sl = pl.ds(j * CHUNK, CHUNK)
        oc = pltpu.make_async_copy(ov.at[sl], out_hbm.at[pl.ds(base + j * CHUNK, CHUNK)], sem)
        oc.start()
        out_copies.append(oc)
    for oc in out_copies:
        oc.wait()


def kernel(user, item, theta_w, a_w, b_w, c_w):
    return _irt_sc(
        user,
        item,
        theta_w.reshape(1, USERS),
        a_w.reshape(1, ITEMS),
        b_w.reshape(1, ITEMS),
        c_w.reshape(1, ITEMS),
    )


# speedup vs baseline: 3.6798x; 1.0024x over previous
import functools

import jax
import jax.numpy as jnp
from jax import lax
from jax.experimental import pallas as pl
from jax.experimental.pallas import tpu as pltpu
from jax.experimental.pallas import tpu_sc as plsc

BATCH = 16384
USERS = 1000000
ITEMS = 100000

NC = 2
NS = 16
L = 16
NW = NC * NS
BPW = BATCH // NW
CHUNK = 128
NCHUNK = BPW // CHUNK
VPC = CHUNK // L

_mesh = plsc.VectorSubcoreMesh(core_axis_name="c", subcore_axis_name="s")


def _softplus(x):
    t = jnp.exp(-jnp.abs(x))
    g = t * (0.99930147 + t * (-0.48463644 + t * (0.25187601 + t * -0.07389941)))
    g = g + (1.0 + t) * jnp.exp(-g) - 1.0
    return jnp.maximum(x, 0.0) + g


@functools.partial(
    pl.kernel,
    out_type=jax.ShapeDtypeStruct((BATCH,), jnp.float32),
    mesh=_mesh,
    scratch_types=[
        pltpu.VMEM((BPW,), jnp.int32),
        pltpu.VMEM((BPW,), jnp.int32),
        pltpu.VMEM((BPW,), jnp.float32),
        pltpu.VMEM((BPW,), jnp.float32),
        pltpu.VMEM((BPW,), jnp.float32),
        pltpu.VMEM((BPW,), jnp.float32),
        pltpu.VMEM((BPW,), jnp.float32),
        pltpu.SemaphoreType.DMA,
        pltpu.SemaphoreType.DMA((NCHUNK,)),
    ],
)
def _irt_sc(user_hbm, item_hbm, theta_hbm, a_hbm, b_hbm, c_hbm, out_hbm,
            uidx, iidx, tv, av, bv, cv, ov, sem, gsem):
    wid = lax.axis_index("s") * NC + lax.axis_index("c")
    base = wid * BPW

    theta_flat = theta_hbm.at[0]
    a_flat = a_hbm.at[0]
    b_flat = b_hbm.at[0]
    c_flat = c_hbm.at[0]

    iu = pltpu.make_async_copy(user_hbm.at[pl.ds(base, BPW)], uidx, sem)
    ii = pltpu.make_async_copy(item_hbm.at[pl.ds(base, BPW)], iidx, sem)
    iu.start()
    ii.start()
    iu.wait()
    ii.wait()

    def _chunk_copies(j):
        sl = pl.ds(j * CHUNK, CHUNK)
        return (
            pltpu.make_async_copy(theta_flat.at[uidx.at[sl]], tv.at[sl], gsem.at[j]),
            pltpu.make_async_copy(a_flat.at[iidx.at[sl]], av.at[sl], gsem.at[j]),
            pltpu.make_async_copy(b_flat.at[iidx.at[sl]], bv.at[sl], gsem.at[j]),
            pltpu.make_async_copy(c_flat.at[iidx.at[sl]], cv.at[sl], gsem.at[j]),
        )

    def fire(j, carry):
        for cpy in _chunk_copies(j):
            cpy.start()
        return carry

    lax.fori_loop(0, NCHUNK, fire, 0)

    def vbody(i, carry):
        s = pl.ds(i * L, L)
        theta = tv[s]
        a = _softplus(av[s])
        b = bv[s]
        c = 1.0 / (1.0 + jnp.exp(-cv[s]))
        ov[s] = c + (1.0 - c) / (1.0 + jnp.exp(-a * (theta - b)))
        return carry

    def consume(j, carry):
        for cpy in _chunk_copies(j):
            cpy.wait()
        lax.fori_loop(j * VPC, (j + 1) * VPC, vbody, 0)
        oc = pltpu.make_async_copy(
            ov.at[pl.ds(j * CHUNK, CHUNK)],
            out_hbm.at[pl.ds(base + j * CHUNK, CHUNK)],
            sem,
        )
        oc.start()
        return carry

    lax.fori_loop(0, NCHUNK, consume, 0)

    def drain(j, carry):
        pltpu.make_async_copy(
            ov.at[pl.ds(j * CHUNK, CHUNK)],
            out_hbm.at[pl.ds(base + j * CHUNK, CHUNK)],
            sem,
        ).wait()
        return carry

    lax.fori_loop(0, NCHUNK, drain, 0)


def kernel(user, item, theta_w, a_w, b_w, c_w):
    return _irt_sc(
        user,
        item,
        theta_w.reshape(1, USERS),
        a_w.reshape(1, ITEMS),
        b_w.reshape(1, ITEMS),
        c_w.reshape(1, ITEMS),
    )
